# baseline (speedup 1.0000x reference)
import functools

import jax
import jax.numpy as jnp
import numpy as np
from jax import lax
from jax.experimental import pallas as pl
from jax.experimental.pallas import tpu as pltpu
from jax.experimental.pallas import tpu_sc as plsc

BIG = np.float32(1e9)
K = 16
N = 4096
NCHUNK = N // 16


def _sort16(x, descending=False):
    k, _ = plsc.sort_key_val(x, x, descending=descending)
    return k


def _merge_top(u, l, xd):
    lo1 = jnp.minimum(u, xd)
    u2 = _sort16(jnp.maximum(u, xd), descending=False)
    hi2 = jnp.maximum(l, _sort16(lo1, descending=True))
    l2 = _sort16(hi2, descending=False)
    return u2, l2


def _valid_mask(e, st):
    inf = np.float32(np.inf)
    fin = (jnp.abs(e) != inf) & (jnp.abs(st) != inf)
    nz = (e * st) != np.float32(0.0)
    df = (e - st) != np.float32(0.0)
    return jnp.where(fin & nz & df, np.float32(1.0), np.float32(0.0))


def _sqrt16(xv):
    bits = plsc.bitcast(xv, jnp.int32)
    y = plsc.bitcast((bits >> 1) + np.int32(0x1FBD1DF5), jnp.float32)
    half = np.float32(0.5)
    for _ in range(4):
        y = half * (y + xv / y)
    return y


def _toploss_body(beta_hbm, ground_hbm, out_hbm, diag_hbm, loss_hbm, img_v,
                  stage_v, s32_v, d_v, g_v, res_v):
    c = lax.axis_index("c")
    s = lax.axis_index("s")
    iota = lax.iota(jnp.int32, 16)

    @pl.when(s < 6)
    def _load_beta():
        pltpu.sync_copy(beta_hbm.at[pl.ds((6 * c + s) * N, N)], img_v)

    @pl.when((s >= 6) & (s < 12))
    def _load_ground():
        pltpu.sync_copy(ground_hbm.at[pl.ds((6 * c + s - 6) * N, N)], img_v)

    @pl.when(s < 12)
    def _phase1():
        x0 = img_v[pl.ds(0, 16)]
        x1 = img_v[pl.ds(16, 16)]
        a = _sort16(x0, descending=False)
        b = _sort16(x1, descending=True)
        u = _sort16(jnp.maximum(a, b), descending=False)
        l = _sort16(jnp.minimum(a, b), descending=False)
        an = _sort16(-x0, descending=False)
        bn = _sort16(-x1, descending=True)
        bu = _sort16(jnp.maximum(an, bn), descending=False)
        bl = _sort16(jnp.minimum(an, bn), descending=False)

        def body(k, carry):
            u, l, bu, bl, tmin, bmin = carry
            x = img_v[pl.ds(k * 16, 16)]

            def do_top(uu, ll):
                u2, l2 = _merge_top(uu, ll, _sort16(x, descending=True))
                return u2, l2, jnp.min(l2)

            def do_bot(uu, ll):
                u2, l2 = _merge_top(uu, ll, _sort16(-x, descending=True))
                return u2, l2, jnp.min(l2)

            u, l, tmin = lax.cond(jnp.max(x) > tmin, do_top,
                                  lambda uu, ll: (uu, ll, tmin), u, l)
            bu, bl, bmin = lax.cond(-jnp.min(x) > bmin, do_bot,
                                    lambda uu, ll: (uu, ll, bmin), bu, bl)
            return u, l, bu, bl, tmin, bmin

        u, l, bu, bl, _, _ = lax.fori_loop(
            2, NCHUNK, body, (u, l, bu, bl, jnp.min(l), jnp.min(bl)))

        s32_v[pl.ds(0, 16)] = _sort16(u, descending=True)
        s32_v[pl.ds(16, 16)] = _sort16(l, descending=True)
        stage_v[pl.ds(0, 16)] = plsc.load_gather(s32_v, [2 * iota])
        stage_v[pl.ds(16, 16)] = plsc.load_gather(s32_v, [2 * iota + 1])
        s32_v[pl.ds(0, 16)] = -_sort16(bu, descending=True)
        s32_v[pl.ds(16, 16)] = -_sort16(bl, descending=True)
        stage_v[pl.ds(32, 16)] = plsc.load_gather(s32_v, [2 * iota + 1])
        stage_v[pl.ds(48, 16)] = plsc.load_gather(s32_v, [2 * iota])
        pltpu.sync_copy(stage_v, diag_hbm.at[pl.ds((12 * c + s) * 64, 64)])

    plsc.subcore_barrier()

    @pl.when(s < 12)
    def _phase2():
        q = s // 6
        sig = s - 6 * q
        pltpu.sync_copy(diag_hbm.at[pl.ds((12 * c + sig) * 64, 64)], d_v)
        pltpu.sync_copy(diag_hbm.at[pl.ds((12 * c + 6 + sig) * 64, 64)], g_v)
        q32 = q * 32
        de = d_v[pl.ds(q32, 16)]
        dst = d_v[pl.ds(q32 + 16, 16)]
        ge = g_v[pl.ds(q32, 16)]
        gs = g_v[pl.ds(q32 + 16, 16)]

        m = _valid_mask(de, dst)
        mg = _valid_mask(ge, gs)
        pen = (np.float32(1.0) - mg) * BIG

        used = jnp.zeros((16,), jnp.float32)
        acc = np.float32(0.0)
        one = np.float32(1.0)
        for i in range(K):
            e_i = de[i]
            s_i = dst[i]
            m_i = m[i]
            dx = e_i - ge
            dy = s_i - gs
            crow = dx * dx + dy * dy + pen + used * BIG
            mn = jnp.min(crow)
            j = plsc.all_reduce_ffs(crow == mn)
            oh = iota == j
            mg_j = jnp.sum(jnp.where(oh, mg, np.float32(0.0)))
            ge_j = jnp.sum(jnp.where(oh, ge, np.float32(0.0)))
            gs_j = jnp.sum(jnp.where(oh, gs, np.float32(0.0)))
            take = m_i * mg_j
            rm = (e_i + s_i) * np.float32(0.5)
            o_e = take * ge_j + (one - take) * rm
            o_s = take * gs_j + (one - take) * rm
            dd_e = (e_i - o_e) * m_i
            dd_s = (s_i - o_s) * m_i
            acc = acc + dd_e * dd_e + dd_s * dd_s
            used = used + jnp.where(oh, take, np.float32(0.0))

        xv = acc + np.float32(1e-12) + jnp.zeros((16,), jnp.float32)
        res_v[...] = _sqrt16(xv)
        pltpu.sync_copy(res_v, loss_hbm.at[pl.ds((12 * c + s) * 16, 16)])

    plsc.subcore_barrier()

    @pl.when(s == 0)
    def _phase3():
        total = jnp.zeros((16,), jnp.float32)
        for w in range(12):
            pltpu.sync_copy(loss_hbm.at[pl.ds((12 * c + w) * 16, 16)], res_v)
            total = total + res_v[...]
        res_v[...] = total * np.float32(1.0 / 12.0)
        pltpu.sync_copy(res_v, out_hbm.at[pl.ds(c * 16, 16)])


@functools.partial(
    pl.kernel,
    out_type=(
        jax.ShapeDtypeStruct((32,), jnp.float32),
        jax.ShapeDtypeStruct((24 * 64,), jnp.float32),
        jax.ShapeDtypeStruct((24 * 16,), jnp.float32),
    ),
    mesh=plsc.VectorSubcoreMesh(core_axis_name="c", subcore_axis_name="s",
                                num_cores=2, num_subcores=16),
    compiler_params=pltpu.CompilerParams(needs_layout_passes=False),
    scratch_types=[
        pltpu.VMEM((N,), jnp.float32),
        pltpu.VMEM((64,), jnp.float32),
        pltpu.VMEM((32,), jnp.float32),
        pltpu.VMEM((64,), jnp.float32),
        pltpu.VMEM((64,), jnp.float32),
        pltpu.VMEM((16,), jnp.float32),
    ],
)
def _toploss(beta_hbm, ground_hbm, out_hbm, diag_hbm, loss_hbm, img_v,
             stage_v, s32_v, d_v, g_v, res_v):
    _toploss_body(beta_hbm, ground_hbm, out_hbm, diag_hbm, loss_hbm, img_v,
                  stage_v, s32_v, d_v, g_v, res_v)


@jax.jit
def kernel(beta, ground):
    out, _, _ = _toploss(beta.reshape(-1), ground.reshape(-1))
    return out[0] + out[16]


# speedup vs baseline: 1.4739x; 1.4739x over previous
import functools

import jax
import jax.numpy as jnp
import numpy as np
from jax import lax
from jax.experimental import pallas as pl
from jax.experimental.pallas import tpu as pltpu
from jax.experimental.pallas import tpu_sc as plsc

BIG = np.float32(1e9)
K = 16
N = 4096
HALF = 128


def _sort16(x, descending=False):
    k, _ = plsc.sort_key_val(x, x, descending=descending)
    return k


def _merge_top(u, l, xd):
    lo1 = jnp.minimum(u, xd)
    u2 = _sort16(jnp.maximum(u, xd), descending=False)
    hi2 = jnp.maximum(l, _sort16(lo1, descending=True))
    l2 = _sort16(hi2, descending=False)
    return u2, l2


def _merge_sets(u0, l0, u1, l1):
    u, l = _merge_top(u0, l0, _sort16(u1, descending=True))
    return _merge_top(u, l, _sort16(l1, descending=True))


def _valid_mask(e, st):
    inf = np.float32(np.inf)
    fin = (jnp.abs(e) != inf) & (jnp.abs(st) != inf)
    nz = (e * st) != np.float32(0.0)
    df = (e - st) != np.float32(0.0)
    return jnp.where(fin & nz & df, np.float32(1.0), np.float32(0.0))


def _sqrt16(xv):
    bits = plsc.bitcast(xv, jnp.int32)
    y = plsc.bitcast((bits >> 1) + np.int32(0x1FBD1DF5), jnp.float32)
    half = np.float32(0.5)
    for _ in range(4):
        y = half * (y + xv / y)
    return y


def _toploss_body(beta_hbm, ground_hbm, out_hbm, loss_hbm, img_v, img2_v,
                  s32_v, loss12_v, res_v):
    c = lax.axis_index("c")
    s = lax.axis_index("s")
    iota = lax.iota(jnp.int32, 16)

    @pl.when(s < 12)
    def _task():
        q = s // 6
        sig = s - 6 * q
        sl = 6 * c + sig
        pltpu.sync_copy(beta_hbm.at[pl.ds(sl * N, N)], img_v)
        pltpu.sync_copy(ground_hbm.at[pl.ds(sl * N, N)], img2_v)
        sgn = jnp.where(q == 0, np.float32(1.0), np.float32(-1.0))

        def chunk(ref, base, k):
            return sgn * ref[pl.ds((base + k) * 16, 16)]

        def init_chain(ref, base):
            a = _sort16(chunk(ref, base, 0), descending=False)
            b = _sort16(chunk(ref, base, 1), descending=True)
            u = _sort16(jnp.maximum(a, b), descending=False)
            l = _sort16(jnp.minimum(a, b), descending=False)
            return u, l

        u00, l00 = init_chain(img_v, 0)
        u01, l01 = init_chain(img_v, HALF)
        u10, l10 = init_chain(img2_v, 0)
        u11, l11 = init_chain(img2_v, HALF)

        def body(k, carry):
            u00, l00, u01, l01, u10, l10, u11, l11 = carry
            u00, l00 = _merge_top(
                u00, l00, _sort16(chunk(img_v, 0, k), descending=True))
            u01, l01 = _merge_top(
                u01, l01, _sort16(chunk(img_v, HALF, k), descending=True))
            u10, l10 = _merge_top(
                u10, l10, _sort16(chunk(img2_v, 0, k), descending=True))
            u11, l11 = _merge_top(
                u11, l11, _sort16(chunk(img2_v, HALF, k), descending=True))
            return u00, l00, u01, l01, u10, l10, u11, l11

        u00, l00, u01, l01, u10, l10, u11, l11 = lax.fori_loop(
            2, HALF, body, (u00, l00, u01, l01, u10, l10, u11, l11))

        ub, lb = _merge_sets(u00, l00, u01, l01)
        ug, lg = _merge_sets(u10, l10, u11, l11)

        idx_e = jnp.where(q == 0, 2 * iota, 2 * iota + 1)
        idx_s = jnp.where(q == 0, 2 * iota + 1, 2 * iota)
        s32_v[pl.ds(0, 16)] = _sort16(ub, descending=True)
        s32_v[pl.ds(16, 16)] = _sort16(lb, descending=True)
        de = sgn * plsc.load_gather(s32_v, [idx_e])
        dst = sgn * plsc.load_gather(s32_v, [idx_s])
        s32_v[pl.ds(0, 16)] = _sort16(ug, descending=True)
        s32_v[pl.ds(16, 16)] = _sort16(lg, descending=True)
        ge = sgn * plsc.load_gather(s32_v, [idx_e])
        gs = sgn * plsc.load_gather(s32_v, [idx_s])

        m = _valid_mask(de, dst)
        mg = _valid_mask(ge, gs)
        pen = (np.float32(1.0) - mg) * BIG

        used = jnp.zeros((16,), jnp.float32)
        acc = np.float32(0.0)
        one = np.float32(1.0)
        for i in range(K):
            e_i = de[i]
            s_i = dst[i]
            m_i = m[i]
            dx = e_i - ge
            dy = s_i - gs
            crow = dx * dx + dy * dy + pen + used * BIG
            mn = jnp.min(crow)
            j = plsc.all_reduce_ffs(crow == mn)
            oh = iota == j
            mg_j = jnp.sum(jnp.where(oh, mg, np.float32(0.0)))
            ge_j = jnp.sum(jnp.where(oh, ge, np.float32(0.0)))
            gs_j = jnp.sum(jnp.where(oh, gs, np.float32(0.0)))
            take = m_i * mg_j
            rm = (e_i + s_i) * np.float32(0.5)
            o_e = take * ge_j + (one - take) * rm
            o_s = take * gs_j + (one - take) * rm
            dd_e = (e_i - o_e) * m_i
            dd_s = (s_i - o_s) * m_i
            acc = acc + dd_e * dd_e + dd_s * dd_s
            used = used + jnp.where(oh, take, np.float32(0.0))

        xv = acc + np.float32(1e-12) + jnp.zeros((16,), jnp.float32)
        res_v[...] = _sqrt16(xv)
        pltpu.sync_copy(res_v, loss_hbm.at[pl.ds((12 * c + s) * 16, 16)])

    plsc.subcore_barrier()

    @pl.when(s == 0)
    def _reduce():
        pltpu.sync_copy(loss_hbm.at[pl.ds(c * 192, 192)], loss12_v)
        total = jnp.zeros((16,), jnp.float32)
        for w in range(12):
            total = total + loss12_v[pl.ds(w * 16, 16)]
        res_v[...] = total * np.float32(1.0 / 12.0)
        pltpu.sync_copy(res_v, out_hbm.at[pl.ds(c * 16, 16)])


@functools.partial(
    pl.kernel,
    out_type=(
        jax.ShapeDtypeStruct((32,), jnp.float32),
        jax.ShapeDtypeStruct((24 * 16,), jnp.float32),
    ),
    mesh=plsc.VectorSubcoreMesh(core_axis_name="c", subcore_axis_name="s",
                                num_cores=2, num_subcores=16),
    compiler_params=pltpu.CompilerParams(needs_layout_passes=False),
    scratch_types=[
        pltpu.VMEM((N,), jnp.float32),
        pltpu.VMEM((N,), jnp.float32),
        pltpu.VMEM((32,), jnp.float32),
        pltpu.VMEM((192,), jnp.float32),
        pltpu.VMEM((16,), jnp.float32),
    ],
)
def _toploss(beta_hbm, ground_hbm, out_hbm, loss_hbm, img_v, img2_v, s32_v,
             loss12_v, res_v):
    _toploss_body(beta_hbm, ground_hbm, out_hbm, loss_hbm, img_v, img2_v,
                  s32_v, loss12_v, res_v)


@jax.jit
def kernel(beta, ground):
    out, _ = _toploss(beta.reshape(-1), ground.reshape(-1))
    return out[0] + out[16]


# speedup vs baseline: 1.4837x; 1.0067x over previous
import functools

import jax
import jax.numpy as jnp
import numpy as np
from jax import lax
from jax.experimental import pallas as pl
from jax.experimental.pallas import tpu as pltpu
from jax.experimental.pallas import tpu_sc as plsc

BIG = np.float32(1e9)
K = 16
N = 4096
HALF = 128


def _sort16(x, descending=False):
    k, _ = plsc.sort_key_val(x, x, descending=descending)
    return k


def _merge_top(u, l, xd):
    lo1 = jnp.minimum(u, xd)
    u2 = _sort16(jnp.maximum(u, xd), descending=False)
    hi2 = jnp.maximum(l, _sort16(lo1, descending=True))
    l2 = _sort16(hi2, descending=False)
    return u2, l2


def _merge_sets(u0, l0, u1, l1):
    u, l = _merge_top(u0, l0, _sort16(u1, descending=True))
    return _merge_top(u, l, _sort16(l1, descending=True))


def _valid_mask(e, st):
    inf = np.float32(np.inf)
    fin = (jnp.abs(e) != inf) & (jnp.abs(st) != inf)
    nz = (e * st) != np.float32(0.0)
    df = (e - st) != np.float32(0.0)
    return jnp.where(fin & nz & df, np.float32(1.0), np.float32(0.0))


def _sqrt16(xv):
    bits = plsc.bitcast(xv, jnp.int32)
    y = plsc.bitcast((bits >> 1) + np.int32(0x1FBD1DF5), jnp.float32)
    half = np.float32(0.5)
    for _ in range(4):
        y = half * (y + xv / y)
    return y


def _toploss_body(beta_hbm, ground_hbm, out_hbm, img_v, img2_v,
                  s32_v, loss12_v, res_v):
    c = lax.axis_index("c")
    s = lax.axis_index("s")
    iota = lax.iota(jnp.int32, 16)

    @pl.when(s < 12)
    def _task():
        q = s // 6
        sig = s - 6 * q
        sl = 6 * c + sig
        pltpu.sync_copy(beta_hbm.at[pl.ds(sl * N, N)], img_v)
        pltpu.sync_copy(ground_hbm.at[pl.ds(sl * N, N)], img2_v)
        sgn = jnp.where(q == 0, np.float32(1.0), np.float32(-1.0))

        def chunk(ref, base, k):
            return sgn * ref[pl.ds((base + k) * 16, 16)]

        def init_chain(ref, base):
            a = _sort16(chunk(ref, base, 0), descending=False)
            b = _sort16(chunk(ref, base, 1), descending=True)
            u = _sort16(jnp.maximum(a, b), descending=False)
            l = _sort16(jnp.minimum(a, b), descending=False)
            return u, l

        u00, l00 = init_chain(img_v, 0)
        u01, l01 = init_chain(img_v, HALF)
        u10, l10 = init_chain(img2_v, 0)
        u11, l11 = init_chain(img2_v, HALF)

        def body(k, carry):
            u00, l00, u01, l01, u10, l10, u11, l11 = carry
            u00, l00 = _merge_top(
                u00, l00, _sort16(chunk(img_v, 0, k), descending=True))
            u01, l01 = _merge_top(
                u01, l01, _sort16(chunk(img_v, HALF, k), descending=True))
            u10, l10 = _merge_top(
                u10, l10, _sort16(chunk(img2_v, 0, k), descending=True))
            u11, l11 = _merge_top(
                u11, l11, _sort16(chunk(img2_v, HALF, k), descending=True))
            return u00, l00, u01, l01, u10, l10, u11, l11

        u00, l00, u01, l01, u10, l10, u11, l11 = lax.fori_loop(
            2, HALF, body, (u00, l00, u01, l01, u10, l10, u11, l11))

        ub, lb = _merge_sets(u00, l00, u01, l01)
        ug, lg = _merge_sets(u10, l10, u11, l11)

        idx_e = jnp.where(q == 0, 2 * iota, 2 * iota + 1)
        idx_s = jnp.where(q == 0, 2 * iota + 1, 2 * iota)
        s32_v[pl.ds(0, 16)] = _sort16(ub, descending=True)
        s32_v[pl.ds(16, 16)] = _sort16(lb, descending=True)
        de = sgn * plsc.load_gather(s32_v, [idx_e])
        dst = sgn * plsc.load_gather(s32_v, [idx_s])
        s32_v[pl.ds(0, 16)] = _sort16(ug, descending=True)
        s32_v[pl.ds(16, 16)] = _sort16(lg, descending=True)
        ge = sgn * plsc.load_gather(s32_v, [idx_e])
        gs = sgn * plsc.load_gather(s32_v, [idx_s])

        m = _valid_mask(de, dst)
        mg = _valid_mask(ge, gs)
        pen = (np.float32(1.0) - mg) * BIG

        used = jnp.zeros((16,), jnp.float32)
        acc = np.float32(0.0)
        one = np.float32(1.0)
        for i in range(K):
            e_i = de[i]
            s_i = dst[i]
            m_i = m[i]
            dx = e_i - ge
            dy = s_i - gs
            crow = dx * dx + dy * dy + pen + used * BIG
            mn = jnp.min(crow)
            j = plsc.all_reduce_ffs(crow == mn)
            oh = iota == j
            mg_j = jnp.sum(jnp.where(oh, mg, np.float32(0.0)))
            ge_j = jnp.sum(jnp.where(oh, ge, np.float32(0.0)))
            gs_j = jnp.sum(jnp.where(oh, gs, np.float32(0.0)))
            take = m_i * mg_j
            rm = (e_i + s_i) * np.float32(0.5)
            o_e = take * ge_j + (one - take) * rm
            o_s = take * gs_j + (one - take) * rm
            dd_e = (e_i - o_e) * m_i
            dd_s = (s_i - o_s) * m_i
            acc = acc + dd_e * dd_e + dd_s * dd_s
            used = used + jnp.where(oh, take, np.float32(0.0))

        xv = acc + np.float32(1e-12) + jnp.zeros((16,), jnp.float32)
        res_v[...] = _sqrt16(xv)
        pltpu.sync_copy(res_v, out_hbm.at[pl.ds(32 + (12 * c + s) * 16, 16)])

    plsc.subcore_barrier()

    @pl.when(s == 0)
    def _reduce():
        pltpu.sync_copy(out_hbm.at[pl.ds(32 + c * 192, 192)], loss12_v)
        total = jnp.zeros((16,), jnp.float32)
        for w in range(12):
            total = total + loss12_v[pl.ds(w * 16, 16)]
        res_v[...] = total * np.float32(1.0 / 12.0)
        pltpu.sync_copy(res_v, out_hbm.at[pl.ds(c * 16, 16)])


@functools.partial(
    pl.kernel,
    out_type=jax.ShapeDtypeStruct((32 + 24 * 16,), jnp.float32),
    mesh=plsc.VectorSubcoreMesh(core_axis_name="c", subcore_axis_name="s",
                                num_cores=2, num_subcores=16),
    compiler_params=pltpu.CompilerParams(needs_layout_passes=False),
    scratch_types=[
        pltpu.VMEM((N,), jnp.float32),
        pltpu.VMEM((N,), jnp.float32),
        pltpu.VMEM((32,), jnp.float32),
        pltpu.VMEM((192,), jnp.float32),
        pltpu.VMEM((16,), jnp.float32),
    ],
)
def _toploss(beta_hbm, ground_hbm, out_hbm, img_v, img2_v, s32_v,
             loss12_v, res_v):
    _toploss_body(beta_hbm, ground_hbm, out_hbm, img_v, img2_v,
                  s32_v, loss12_v, res_v)


@jax.jit
def kernel(beta, ground):
    out = _toploss(beta.reshape(-1), ground.reshape(-1))
    return out[0] + out[16]
